# Initial kernel scaffold; baseline (speedup 1.0000x reference)
import jax
import jax.numpy as jnp
from jax.experimental import pallas as pl


def kernel(token_indices, special_table, blosum_table):
    raise NotImplementedError("write your pallas kernel here")


# speedup vs baseline: 5.2874x; 5.2874x over previous
import functools

import jax
import jax.numpy as jnp
from jax import lax
from jax.experimental import pallas as pl
from jax.experimental.pallas import tpu as pltpu
from jax.experimental.pallas import tpu_sc as plsc

_NC, _NS = 2, 16
_NW = _NC * _NS
_L = 16
_CHUNK = 2048


def _combine_tables(special_table, blosum_table):
    n_special = special_table.shape[0]

    def body(sp_ref, bl_ref, out_ref):
        bl = bl_ref[...]
        rid = lax.broadcasted_iota(jnp.int32, bl.shape, 0)
        ext = jnp.broadcast_to(sp_ref[0:1], bl.shape)
        for r in range(1, n_special):
            row = jnp.broadcast_to(sp_ref[r:r + 1], bl.shape)
            ext = jnp.where(rid == r, row, ext)
        out_ref[...] = bl + ext

    return pl.pallas_call(
        body,
        out_shape=jax.ShapeDtypeStruct(blosum_table.shape, blosum_table.dtype),
    )(special_table, blosum_table)


@functools.partial(jax.jit, static_argnames=("n", "d"))
def _sc_gather(comb_flat, idx_flat, n, d):
    vd = comb_flat.shape[0]
    tokens_per_worker = n // _NW
    n_chunks = tokens_per_worker // _CHUNK
    assert n_chunks % 2 == 0
    mesh = plsc.VectorSubcoreMesh(core_axis_name="c", subcore_axis_name="s")

    @functools.partial(
        pl.kernel,
        out_type=jax.ShapeDtypeStruct((n * d,), jnp.float32),
        mesh=mesh,
        scratch_types=[
            pltpu.VMEM((vd,), jnp.float32),
            pltpu.VMEM((_CHUNK,), jnp.int32),
            pltpu.VMEM((_CHUNK,), jnp.int32),
            pltpu.VMEM((_CHUNK * d,), jnp.float32),
            pltpu.VMEM((_CHUNK * d,), jnp.float32),
            pltpu.SemaphoreType.DMA,
            pltpu.SemaphoreType.DMA,
        ],
        compiler_params=pltpu.CompilerParams(
            use_tc_tiling_on_sc=False, needs_layout_passes=False),
    )
    def k(comb_hbm, idx_hbm, out_hbm, comb_v, idx_v0, idx_v1, pk_v0, pk_v1,
          sem0, sem1):
        idx_v = (idx_v0, idx_v1)
        pk_v = (pk_v0, pk_v1)
        sem = (sem0, sem1)
        wid = lax.axis_index("c") * _NS + lax.axis_index("s")
        wbase = wid * tokens_per_worker
        pltpu.sync_copy(comb_hbm, comb_v)
        lanes_d = lax.iota(jnp.int32, _L) * d

        def chunk(pi, carry):
            for slot in range(2):
                ci = pi * 2 + slot
                tbase = wbase + ci * _CHUNK

                @pl.when(pi > 0)
                def _():
                    pltpu.make_async_copy(
                        pk_v[slot], out_hbm.at[pl.ds(0, _CHUNK * d)],
                        sem[slot]).wait()

                pltpu.sync_copy(idx_hbm.at[pl.ds(tbase, _CHUNK)], idx_v[slot])
                idx_ref = idx_v[slot]
                pk_ref = pk_v[slot]

                def grp(g, c):
                    idxv = idx_ref[pl.ds(g * _L, _L)]
                    avec = idxv * d
                    pvec = lanes_d + g * (_L * d)
                    for dd in range(d):
                        v = plsc.load_gather(comb_v, [avec + dd])
                        plsc.store_scatter(pk_ref, [pvec + dd], v)
                    return c

                lax.fori_loop(0, _CHUNK // _L, grp, 0)
                pltpu.async_copy(
                    pk_ref, out_hbm.at[pl.ds(tbase * d, _CHUNK * d)],
                    sem[slot])
            return carry

        lax.fori_loop(0, n_chunks // 2, chunk, 0)
        for slot in range(2):
            pltpu.make_async_copy(
                pk_v[slot], out_hbm.at[pl.ds(0, _CHUNK * d)], sem[slot]).wait()

    return k(comb_flat, idx_flat)


def kernel(token_indices, special_table, blosum_table):
    b, s = token_indices.shape
    d = blosum_table.shape[1]
    n = b * s
    assert n % (_NW * _CHUNK * 2) == 0
    comb = _combine_tables(special_table, blosum_table)
    idx_flat = token_indices.astype(jnp.int32).reshape(n)
    out = _sc_gather(comb.reshape(-1), idx_flat, n, d)
    return out.reshape(b, s, d)


# speedup vs baseline: 5.8506x; 1.1065x over previous
import functools

import jax
import jax.numpy as jnp
from jax import lax
from jax.experimental import pallas as pl
from jax.experimental.pallas import tpu as pltpu
from jax.experimental.pallas import tpu_sc as plsc

_NC, _NS = 2, 16
_NW = _NC * _NS
_L = 16
_CHUNK = 2048


def _combine_tables(special_table, blosum_table):
    n_special = special_table.shape[0]

    def body(sp_ref, bl_ref, out_ref):
        bl = bl_ref[...]
        rid = lax.broadcasted_iota(jnp.int32, bl.shape, 0)
        ext = jnp.broadcast_to(sp_ref[0:1], bl.shape)
        for r in range(1, n_special):
            row = jnp.broadcast_to(sp_ref[r:r + 1], bl.shape)
            ext = jnp.where(rid == r, row, ext)
        out_ref[...] = bl + ext

    return pl.pallas_call(
        body,
        out_shape=jax.ShapeDtypeStruct(blosum_table.shape, blosum_table.dtype),
    )(special_table, blosum_table)


@functools.partial(jax.jit, static_argnames=("n", "d"))
def _sc_gather(comb_flat, idx_flat, n, d):
    vd = comb_flat.shape[0]
    tokens_per_worker = n // _NW
    n_chunks = tokens_per_worker // _CHUNK
    assert n_chunks % 2 == 0
    mesh = plsc.VectorSubcoreMesh(core_axis_name="c", subcore_axis_name="s")

    @functools.partial(
        pl.kernel,
        out_type=jax.ShapeDtypeStruct((n * d,), jnp.float32),
        mesh=mesh,
        scratch_types=[
            pltpu.VMEM((vd,), jnp.float32),
            pltpu.VMEM((_CHUNK,), jnp.int32),
            pltpu.VMEM((_CHUNK,), jnp.int32),
            pltpu.VMEM((_CHUNK * d,), jnp.float32),
            pltpu.VMEM((_CHUNK * d,), jnp.float32),
            pltpu.SemaphoreType.DMA,
            pltpu.SemaphoreType.DMA,
        ],
        compiler_params=pltpu.CompilerParams(
            use_tc_tiling_on_sc=False, needs_layout_passes=False),
    )
    def k(comb_hbm, idx_hbm, out_hbm, comb_v, idx_v0, idx_v1, pk_v0, pk_v1,
          sem0, sem1):
        idx_v = (idx_v0, idx_v1)
        pk_v = (pk_v0, pk_v1)
        sem = (sem0, sem1)
        wid = lax.axis_index("c") * _NS + lax.axis_index("s")
        wbase = wid * tokens_per_worker
        pltpu.sync_copy(comb_hbm, comb_v)
        lanes_d = lax.iota(jnp.int32, _L) * d

        def chunk(pi, carry):
            for slot in range(2):
                ci = pi * 2 + slot
                tbase = wbase + ci * _CHUNK

                @pl.when(pi > 0)
                def _():
                    pltpu.make_async_copy(
                        pk_v[slot], out_hbm.at[pl.ds(0, _CHUNK * d)],
                        sem[slot]).wait()

                pltpu.sync_copy(idx_hbm.at[pl.ds(tbase, _CHUNK)], idx_v[slot])
                idx_ref = idx_v[slot]
                pk_ref = pk_v[slot]

                @plsc.parallel_loop(0, _CHUNK // _L, step=1, unroll=4)
                def _(g):
                    idxv = idx_ref[pl.ds(g * _L, _L)]
                    avec = idxv * d
                    pvec = lanes_d + g * (_L * d)
                    for dd in range(d):
                        v = plsc.load_gather(comb_v, [avec + dd])
                        plsc.store_scatter(pk_ref, [pvec + dd], v)
                pltpu.async_copy(
                    pk_ref, out_hbm.at[pl.ds(tbase * d, _CHUNK * d)],
                    sem[slot])
            return carry

        lax.fori_loop(0, n_chunks // 2, chunk, 0)
        for slot in range(2):
            pltpu.make_async_copy(
                pk_v[slot], out_hbm.at[pl.ds(0, _CHUNK * d)], sem[slot]).wait()

    return k(comb_flat, idx_flat)


def kernel(token_indices, special_table, blosum_table):
    b, s = token_indices.shape
    d = blosum_table.shape[1]
    n = b * s
    assert n % (_NW * _CHUNK * 2) == 0
    comb = _combine_tables(special_table, blosum_table)
    idx_flat = token_indices.astype(jnp.int32).reshape(n)
    out = _sc_gather(comb.reshape(-1), idx_flat, n, d)
    return out.reshape(b, s, d)


# speedup vs baseline: 11.0781x; 1.8935x over previous
import functools

import jax
import jax.numpy as jnp
from jax import lax
from jax.experimental import pallas as pl
from jax.experimental.pallas import tpu as pltpu

_K = 128
_DP = 128
_BB = 8


def _combine_pad_tables(special_table, blosum_table):
    n_special, d = special_table.shape
    v = blosum_table.shape[0]

    def body(sp_ref, bl_ref, out_ref):
        bl = bl_ref[...]
        rid = lax.broadcasted_iota(jnp.int32, bl.shape, 0)
        ext = jnp.broadcast_to(sp_ref[0:1], bl.shape)
        for r in range(1, n_special):
            row = jnp.broadcast_to(sp_ref[r:r + 1], bl.shape)
            ext = jnp.where(rid == r, row, ext)
        comb = bl + ext
        comb = jnp.concatenate(
            [comb, jnp.zeros((v, _DP - d), jnp.float32)], axis=1)
        comb = jnp.concatenate(
            [comb, jnp.zeros((_K - v, _DP), jnp.float32)], axis=0)
        out_ref[...] = comb

    return pl.pallas_call(
        body,
        out_shape=jax.ShapeDtypeStruct((_K, _DP), jnp.float32),
    )(special_table, blosum_table)


def _gather_matmul(idx, w, d):
    b, s = idx.shape

    def body(idx_ref, w_ref, out_ref):
        wmat = w_ref[...]
        for bb in range(_BB):
            row = idx_ref[bb:bb + 1, :]
            oh = (jnp.broadcast_to(row, (_K, s)) ==
                  lax.broadcasted_iota(jnp.int32, (_K, s), 0)
                  ).astype(jnp.float32)
            res = lax.dot_general(
                oh, wmat, (((0,), (0,)), ((), ())),
                preferred_element_type=jnp.float32)
            out_ref[bb] = res[:, :d]

    return pl.pallas_call(
        body,
        grid=(b // _BB,),
        in_specs=[
            pl.BlockSpec((_BB, s), lambda i: (i, 0)),
            pl.BlockSpec((_K, _DP), lambda i: (0, 0)),
        ],
        out_specs=pl.BlockSpec((_BB, s, d), lambda i: (i, 0, 0)),
        out_shape=jax.ShapeDtypeStruct((b, s, d), jnp.float32),
    )(idx, w)


def kernel(token_indices, special_table, blosum_table):
    d = blosum_table.shape[1]
    w = _combine_pad_tables(special_table, blosum_table)
    return _gather_matmul(token_indices.astype(jnp.int32), w, d)


# speedup vs baseline: 64.2353x; 5.7984x over previous
import functools

import jax
import jax.numpy as jnp
from jax import lax
from jax.experimental import pallas as pl
from jax.experimental.pallas import tpu as pltpu

_BS = 8
_BL = 2048
_LANE = 128


def _combine_tables(special_table, blosum_table):
    n_special = special_table.shape[0]

    def body(sp_ref, bl_ref, out_ref):
        bl = bl_ref[...]
        rid = lax.broadcasted_iota(jnp.int32, bl.shape, 0)
        ext = jnp.broadcast_to(sp_ref[0:1], bl.shape)
        for r in range(1, n_special):
            row = jnp.broadcast_to(sp_ref[r:r + 1], bl.shape)
            ext = jnp.where(rid == r, row, ext)
        out_ref[...] = bl + ext

    return pl.pallas_call(
        body,
        out_shape=jax.ShapeDtypeStruct(blosum_table.shape, blosum_table.dtype),
    )(special_table, blosum_table)


def _gather_physical(comb, idx_t):
    v, d = comb.shape
    s, b = idx_t.shape

    def body(comb_ref, idx_ref, out_ref):
        c0 = [comb_ref[0, dd] for dd in range(d)]
        diff = [[comb_ref[k, dd] - c0[dd] for dd in range(d)]
                for k in range(1, v)]
        for c in range(_BL // _LANE):
            t = idx_ref[:, c * _LANE:(c + 1) * _LANE]
            mks = [(t == k).astype(jnp.float32) for k in range(1, v)]
            for dd in range(d):
                acc = jnp.full(t.shape, c0[dd], jnp.float32)
                for ki in range(v - 1):
                    acc = acc + mks[ki] * diff[ki][dd]
                out_ref[dd, :, c * _LANE:(c + 1) * _LANE] = acc

    return pl.pallas_call(
        body,
        grid=(s // _BS, b // _BL),
        in_specs=[
            pl.BlockSpec(memory_space=pltpu.SMEM),
            pl.BlockSpec((_BS, _BL), lambda i, j: (i, j)),
        ],
        out_specs=pl.BlockSpec((d, _BS, _BL), lambda i, j: (0, i, j)),
        out_shape=jax.ShapeDtypeStruct((d, s, b), jnp.float32),
    )(comb, idx_t)


def kernel(token_indices, special_table, blosum_table):
    comb = _combine_tables(special_table, blosum_table)
    idx_t = token_indices.astype(jnp.int32).T
    out_t = _gather_physical(comb, idx_t)
    return out_t.transpose(2, 1, 0)


# speedup vs baseline: 159.6880x; 2.4860x over previous
import functools

import jax
import jax.numpy as jnp
from jax import lax
from jax.experimental import pallas as pl
from jax.experimental.pallas import tpu as pltpu

_BS = 8
_BL = 2048
_LANE = 128


def _combine_tables(special_table, blosum_table):
    n_special = special_table.shape[0]
    v, d = blosum_table.shape

    def body(sp_ref, bl_ref, out_ref):
        bl = bl_ref[...]
        rid = lax.broadcasted_iota(jnp.int32, bl.shape, 0)
        ext = jnp.broadcast_to(sp_ref[0:1], bl.shape)
        for r in range(1, n_special):
            row = jnp.broadcast_to(sp_ref[r:r + 1], bl.shape)
            ext = jnp.where(rid == r, row, ext)
        comb = bl + ext
        comb = jnp.concatenate(
            [comb, jnp.zeros((32 - v, d), jnp.float32)], axis=0)
        for dd in range(d):
            col = comb[:, dd:dd + 1]
            out_ref[dd] = jnp.broadcast_to(col, (32, _LANE))

    return pl.pallas_call(
        body,
        out_shape=jax.ShapeDtypeStruct((d, 32, _LANE), jnp.float32),
    )(special_table, blosum_table)


def _gather_physical(lut, idx_t):
    d = lut.shape[0]
    s, b = idx_t.shape

    def body(lut_ref, idx_ref, out_ref):
        for c in range(_BL // _LANE):
            t = idx_ref[:, c * _LANE:(c + 1) * _LANE]
            tl = jnp.bitwise_and(t, 7)
            hi0 = t < 8
            hi1 = t < 16
            for dd in range(d):
                gs = [
                    jnp.take_along_axis(
                        lut_ref[dd, 8 * g:8 * (g + 1)], tl, axis=0,
                        mode=lax.GatherScatterMode.PROMISE_IN_BOUNDS)
                    for g in range(3)
                ]
                res = jnp.where(hi0, gs[0], jnp.where(hi1, gs[1], gs[2]))
                out_ref[dd, :, c * _LANE:(c + 1) * _LANE] = res

    return pl.pallas_call(
        body,
        grid=(s // _BS, b // _BL),
        in_specs=[
            pl.BlockSpec((d, 32, _LANE), lambda i, j: (0, 0, 0)),
            pl.BlockSpec((_BS, _BL), lambda i, j: (i, j)),
        ],
        out_specs=pl.BlockSpec((d, _BS, _BL), lambda i, j: (0, i, j)),
        out_shape=jax.ShapeDtypeStruct((d, s, b), jnp.float32),
    )(lut, idx_t)


def kernel(token_indices, special_table, blosum_table):
    lut = _combine_tables(special_table, blosum_table)
    idx_t = token_indices.astype(jnp.int32).T
    out_t = _gather_physical(lut, idx_t)
    return out_t.transpose(2, 1, 0)
